# Initial kernel scaffold; baseline (speedup 1.0000x reference)
import jax
import jax.numpy as jnp
from jax.experimental import pallas as pl


def kernel(x, edge_index, W1, b1, W2, b2):
    raise NotImplementedError("write your pallas kernel here")


# speedup vs baseline: 28.7990x; 28.7990x over previous
import functools

import jax
import jax.numpy as jnp
from jax import lax
from jax.experimental import pallas as pl
from jax.experimental.pallas import tpu as pltpu
from jax.experimental.pallas import tpu_sc as plsc

N_NODES = 10000
NP = 10240
N_PAD_ROWS = NP - N_NODES
E = 320000
EP = 327680
NW = 32
NCHUNK = 80
EPW = NCHUNK * 128
STRIPE = NP // 16

_mesh = plsc.VectorSubcoreMesh(core_axis_name="c", subcore_axis_name="s")


def _zero_1d(ref, n):
    z = jnp.zeros((16,), jnp.float32)

    def body(i, _):
        ref[pl.ds(i * 16, 16)] = z
        return 0

    lax.fori_loop(0, n // 16, body, 0)


def _zero_2d(ref, rows):
    z = jnp.zeros((16,), jnp.float32)

    def body(i, _):
        for k in range(8):
            ref[i, pl.ds(k * 16, 16)] = z
        return 0

    lax.fori_loop(0, rows, body, 0)


@functools.partial(
    pl.kernel,
    out_type=jax.ShapeDtypeStruct((2, NP), jnp.float32),
    mesh=_mesh,
    scratch_types=[
        pltpu.VMEM((NCHUNK, 128), jnp.int32),
        pltpu.VMEM((128,), jnp.float32),
        pltpu.VMEM((STRIPE,), jnp.float32),
        pltpu.VMEM_SHARED((NP,), jnp.float32),
    ],
    compiler_params=pltpu.CompilerParams(needs_layout_passes=False),
)
def _deg_kernel(dst_hbm, deg_out, dst_v, ones_v, z_v, deg_s):
    cid = lax.axis_index("c")
    sid = lax.axis_index("s")
    wid = sid * 2 + cid
    pltpu.sync_copy(dst_hbm.at[wid], dst_v)
    one = jnp.ones((16,), jnp.float32)
    for k in range(8):
        ones_v[pl.ds(k * 16, 16)] = one
    _zero_1d(z_v, STRIPE)
    pltpu.sync_copy(z_v, deg_s.at[pl.ds(sid * STRIPE, STRIPE)])
    plsc.subcore_barrier()

    def body(j, _):
        pltpu.sync_copy(ones_v, deg_s.at[dst_v.at[j]], add=True)
        return 0

    lax.fori_loop(0, NCHUNK, body, 0)
    plsc.subcore_barrier()
    pltpu.sync_copy(deg_s.at[pl.ds(sid * STRIPE, STRIPE)],
                    deg_out.at[cid, pl.ds(sid * STRIPE, STRIPE)])


def _scale_body(x_ref, degt_ref, w1_ref, g_ref, dinv_ref):
    deg = degt_ref[:, 0:1] + degt_ref[:, 1:2] + 1.0
    dinv = lax.rsqrt(deg)
    h = jax.lax.dot_general(
        x_ref[...], w1_ref[...], (((1,), (0,)), ((), ())),
        preferred_element_type=jnp.float32,
        precision=lax.Precision.HIGHEST,
    )
    g_ref[...] = dinv * h
    dinv_ref[...] = dinv


def _run_scale(x_p, degt, W1):
    return pl.pallas_call(
        _scale_body,
        grid=(NP // 128,),
        in_specs=[
            pl.BlockSpec((128, 128), lambda i: (i, 0)),
            pl.BlockSpec((128, 2), lambda i: (i, 0)),
            pl.BlockSpec((128, 128), lambda i: (0, 0)),
        ],
        out_specs=[
            pl.BlockSpec((128, 128), lambda i: (i, 0)),
            pl.BlockSpec((128, 1), lambda i: (i, 0)),
        ],
        out_shape=[
            jax.ShapeDtypeStruct((NP, 128), jnp.float32),
            jax.ShapeDtypeStruct((NP, 1), jnp.float32),
        ],
    )(x_p, degt, W1)


@functools.partial(
    pl.kernel,
    out_type=(
        jax.ShapeDtypeStruct((2, NP, 128), jnp.float32),
        jax.ShapeDtypeStruct((2, NP), jnp.float32),
    ),
    mesh=_mesh,
    scratch_types=[
        pltpu.VMEM((NCHUNK, 128), jnp.int32),
        pltpu.VMEM((NCHUNK, 128), jnp.int32),
        pltpu.VMEM((NP,), jnp.float32),
        pltpu.VMEM((128,), jnp.float32),
        pltpu.VMEM((128, 128), jnp.float32),
        pltpu.VMEM((STRIPE,), jnp.float32),
        pltpu.VMEM_SHARED((NP, 128), jnp.float32),
        pltpu.VMEM_SHARED((NP,), jnp.float32),
        pltpu.SemaphoreType.DMA,
    ],
    compiler_params=pltpu.CompilerParams(needs_layout_passes=False),
)
def _agg_kernel(src_hbm, dst_hbm, g_hbm, dinv_hbm,
                acc_out, s_out,
                src_v, dst_v, dinv_v, sval_v, rows_v, z_v,
                acc_s, s_s, sem):
    cid = lax.axis_index("c")
    sid = lax.axis_index("s")
    wid = sid * 2 + cid
    pltpu.sync_copy(src_hbm.at[wid], src_v)
    pltpu.sync_copy(dst_hbm.at[wid], dst_v)
    pltpu.sync_copy(dinv_hbm, dinv_v)
    _zero_2d(rows_v, 128)
    for t in range(STRIPE // 128):
        pltpu.sync_copy(rows_v, acc_s.at[pl.ds(sid * STRIPE + t * 128, 128)])
    _zero_1d(z_v, STRIPE)
    pltpu.sync_copy(z_v, s_s.at[pl.ds(sid * STRIPE, STRIPE)])
    plsc.subcore_barrier()

    def body(j, _):
        pltpu.async_copy(g_hbm.at[src_v.at[j]], rows_v, sem).wait()
        for k in range(8):
            di = dst_v[j, pl.ds(k * 16, 16)]
            sval_v[pl.ds(k * 16, 16)] = plsc.load_gather(dinv_v, [di])
        pltpu.sync_copy(rows_v, acc_s.at[dst_v.at[j]], add=True)
        pltpu.sync_copy(sval_v, s_s.at[src_v.at[j]], add=True)
        return 0

    lax.fori_loop(0, NCHUNK, body, 0)
    plsc.subcore_barrier()
    pltpu.sync_copy(acc_s.at[pl.ds(sid * STRIPE, STRIPE)],
                    acc_out.at[cid, pl.ds(sid * STRIPE, STRIPE)])
    pltpu.sync_copy(s_s.at[pl.ds(sid * STRIPE, STRIPE)],
                    s_out.at[cid, pl.ds(sid * STRIPE, STRIPE)])


def _final_body(accp_ref, g_ref, st_ref, dinv_ref, b1_ref, w2_ref, b2_ref,
                out_ref, vacc_ref):
    i = pl.program_id(0)
    dinv = dinv_ref[...]
    s = st_ref[:, 0:1] + st_ref[:, 1:2]
    c = dinv * s + dinv * dinv
    ids = i * 128 + lax.broadcasted_iota(jnp.int32, (128, 1), 0)
    c = jnp.where(ids < N_NODES, c, 0.0)
    acc = accp_ref[0] + accp_ref[1] + g_ref[...]
    r = jnp.maximum(dinv * acc + b1_ref[...], 0.0)
    part = jnp.sum(c * r, axis=0, keepdims=True)

    @pl.when(i == 0)
    def _():
        vacc_ref[...] = part

    @pl.when(i > 0)
    def _():
        vacc_ref[...] += part

    @pl.when(i == NP // 128 - 1)
    def _():
        v = vacc_ref[...]
        out_ref[...] = (
            jax.lax.dot_general(
                v, w2_ref[...], (((1,), (0,)), ((), ())),
                preferred_element_type=jnp.float32,
                precision=lax.Precision.HIGHEST,
            ) / float(N_NODES) + b2_ref[...]
        )


def _run_final(accp, g, st, dinv, b1, W2, b2):
    return pl.pallas_call(
        _final_body,
        grid=(NP // 128,),
        in_specs=[
            pl.BlockSpec((2, 128, 128), lambda i: (0, i, 0)),
            pl.BlockSpec((128, 128), lambda i: (i, 0)),
            pl.BlockSpec((128, 2), lambda i: (i, 0)),
            pl.BlockSpec((128, 1), lambda i: (i, 0)),
            pl.BlockSpec((1, 128), lambda i: (0, 0)),
            pl.BlockSpec((128, 128), lambda i: (0, 0)),
            pl.BlockSpec((1, 128), lambda i: (0, 0)),
        ],
        out_specs=pl.BlockSpec((1, 128), lambda i: (0, 0)),
        out_shape=jax.ShapeDtypeStruct((1, 128), jnp.float32),
        scratch_shapes=[pltpu.VMEM((1, 128), jnp.float32)],
    )(accp, g, st, dinv, b1, W2, b2)


def kernel(x, edge_index, W1, b1, W2, b2):
    ei = edge_index.astype(jnp.int32)
    pad = N_NODES + (jnp.arange(EP - E, dtype=jnp.int32) % N_PAD_ROWS)
    src = jnp.concatenate([ei[0], pad])
    dst = jnp.concatenate([ei[1], pad])
    src3 = src.reshape(NW, NCHUNK, 128)
    dst3 = dst.reshape(NW, NCHUNK, 128)
    x_p = jnp.concatenate([x, jnp.zeros((NP - N_NODES, 128), x.dtype)])

    degp = _deg_kernel(dst3)
    degt = degp.T.reshape(NP, 2)
    g, dinv = _run_scale(x_p, degt, W1)
    accp, sp = _agg_kernel(src3, dst3, g, dinv.reshape(NP))
    st = sp.T.reshape(NP, 2)
    out = _run_final(accp, g, st, dinv, b1.reshape(1, 128), W2,
                     b2.reshape(1, 128))
    return out.reshape(128)


# speedup vs baseline: 37.3682x; 1.2976x over previous
import functools

import jax
import jax.numpy as jnp
from jax import lax
from jax.experimental import pallas as pl
from jax.experimental.pallas import tpu as pltpu
from jax.experimental.pallas import tpu_sc as plsc

N_NODES = 10000
NP = 10240
N_PAD_ROWS = NP - N_NODES
E = 320000
EP = 327680
NW = 32
CH = 128
NCHUNK = 80
EPW = NCHUNK * CH
STRIPE = NP // 16

_mesh = plsc.VectorSubcoreMesh(core_axis_name="c", subcore_axis_name="s")


def _zero_1d(ref, n):
    z = jnp.zeros((16,), jnp.float32)

    def body(i, _):
        ref[pl.ds(i * 16, 16)] = z
        return 0

    lax.fori_loop(0, n // 16, body, 0)


def _zero_2d(ref, rows):
    z = jnp.zeros((16,), jnp.float32)

    def body(i, _):
        for k in range(8):
            ref[i, pl.ds(k * 16, 16)] = z
        return 0

    lax.fori_loop(0, rows, body, 0)


@functools.partial(
    pl.kernel,
    out_type=jax.ShapeDtypeStruct((2, NP), jnp.float32),
    mesh=_mesh,
    scratch_types=[
        pltpu.VMEM((NCHUNK, CH), jnp.int32),
        pltpu.VMEM((CH,), jnp.float32),
        pltpu.VMEM((STRIPE,), jnp.float32),
        pltpu.VMEM_SHARED((NP,), jnp.float32),
    ],
    compiler_params=pltpu.CompilerParams(needs_layout_passes=False),
)
def _deg_kernel(dst_hbm, deg_out, dst_v, ones_v, z_v, deg_s):
    cid = lax.axis_index("c")
    sid = lax.axis_index("s")
    wid = sid * 2 + cid
    pltpu.sync_copy(dst_hbm.at[wid], dst_v)
    one = jnp.ones((16,), jnp.float32)
    for k in range(CH // 16):
        ones_v[pl.ds(k * 16, 16)] = one
    _zero_1d(z_v, STRIPE)
    pltpu.sync_copy(z_v, deg_s.at[pl.ds(sid * STRIPE, STRIPE)])
    plsc.subcore_barrier()

    def body(j, _):
        pltpu.sync_copy(ones_v, deg_s.at[dst_v.at[j]], add=True)
        return 0

    lax.fori_loop(0, NCHUNK, body, 0)
    plsc.subcore_barrier()
    pltpu.sync_copy(deg_s.at[pl.ds(sid * STRIPE, STRIPE)],
                    deg_out.at[cid, pl.ds(sid * STRIPE, STRIPE)])


def _scale_body(x_ref, degt_ref, w1_ref, g_ref, dinv_ref):
    deg = degt_ref[:, 0:1] + degt_ref[:, 1:2] + 1.0
    dinv = lax.rsqrt(deg)
    h = jax.lax.dot_general(
        x_ref[...], w1_ref[...], (((1,), (0,)), ((), ())),
        preferred_element_type=jnp.float32,
        precision=lax.Precision.HIGHEST,
    )
    g_ref[...] = dinv * h
    dinv_ref[...] = dinv


def _run_scale(x_p, degt, W1):
    return pl.pallas_call(
        _scale_body,
        grid=(NP // 128,),
        in_specs=[
            pl.BlockSpec((128, 128), lambda i: (i, 0)),
            pl.BlockSpec((128, 2), lambda i: (i, 0)),
            pl.BlockSpec((128, 128), lambda i: (0, 0)),
        ],
        out_specs=[
            pl.BlockSpec((128, 128), lambda i: (i, 0)),
            pl.BlockSpec((128, 1), lambda i: (i, 0)),
        ],
        out_shape=[
            jax.ShapeDtypeStruct((NP, 128), jnp.float32),
            jax.ShapeDtypeStruct((NP, 1), jnp.float32),
        ],
    )(x_p, degt, W1)


@functools.partial(
    pl.kernel,
    out_type=(
        jax.ShapeDtypeStruct((2, NP, 128), jnp.float32),
        jax.ShapeDtypeStruct((2, NP), jnp.float32),
    ),
    mesh=_mesh,
    scratch_types=[
        pltpu.VMEM((4, 2, CH), jnp.int32),
        pltpu.VMEM((NP,), jnp.float32),
        pltpu.VMEM((CH,), jnp.float32),
        pltpu.VMEM((CH, 128), jnp.float32),
        pltpu.VMEM((CH, 128), jnp.float32),
        pltpu.VMEM_SHARED((NP, 128), jnp.float32),
        pltpu.VMEM_SHARED((NP,), jnp.float32),
        pltpu.SemaphoreType.DMA,
        pltpu.SemaphoreType.DMA,
        pltpu.SemaphoreType.DMA,
        pltpu.SemaphoreType.DMA,
        pltpu.SemaphoreType.DMA,
        pltpu.SemaphoreType.DMA,
    ],
    compiler_params=pltpu.CompilerParams(needs_layout_passes=False),
)
def _agg_kernel(si_hbm, g_hbm, dinv_hbm,
                acc_out, s_out,
                sd_v, dinv_v, sval_v, rows0_v, rows1_v,
                acc_s, s_s,
                isem0, isem1, isem2, isem3, gsem0, gsem1):
    cid = lax.axis_index("c")
    sid = lax.axis_index("s")
    wid = sid * 2 + cid
    isems = (isem0, isem1, isem2, isem3)
    rows = (rows0_v, rows1_v)
    gsems = (gsem0, gsem1)
    pltpu.sync_copy(dinv_hbm, dinv_v)
    _zero_2d(rows0_v, CH)
    for t in range(STRIPE // CH):
        pltpu.sync_copy(rows0_v, acc_s.at[pl.ds(sid * STRIPE + t * CH, CH)])
    _zero_1d(sval_v, CH)
    for t in range(STRIPE // CH):
        pltpu.sync_copy(sval_v, s_s.at[pl.ds(sid * STRIPE + t * CH, CH)])
    plsc.subcore_barrier()

    def idx_fetch(c, slot):
        pltpu.async_copy(si_hbm.at[wid, c], sd_v.at[slot], isems[slot])

    def idx_wait(c, slot):
        pltpu.make_async_copy(si_hbm.at[wid, c], sd_v.at[slot],
                              isems[slot]).wait()

    def gather_start(c, slot, buf):
        pltpu.async_copy(g_hbm.at[sd_v.at[slot, 0]], rows[buf], gsems[buf])

    def gather_wait(c, slot, buf):
        pltpu.make_async_copy(g_hbm.at[sd_v.at[slot, 0]], rows[buf],
                              gsems[buf]).wait()

    def s_part(slot):
        for k in range(CH // 16):
            di = sd_v[slot, 1, pl.ds(k * 16, 16)]
            sval_v[pl.ds(k * 16, 16)] = plsc.load_gather(dinv_v, [di])

    def consume(c, slot, buf):
        s_part(slot)
        gather_wait(c, slot, buf)
        pltpu.sync_copy(rows[buf], acc_s.at[sd_v.at[slot, 1]], add=True)
        pltpu.sync_copy(sval_v, s_s.at[sd_v.at[slot, 0]], add=True)

    for c in range(4):
        idx_fetch(c, c)
    idx_wait(0, 0)
    gather_start(0, 0, 0)

    def body(t, _):
        c0 = 4 * t
        not_last = t < NCHUNK // 4 - 1
        idx_wait(c0 + 1, 1)
        gather_start(c0 + 1, 1, 1)
        consume(c0, 0, 0)

        @pl.when(not_last)
        def _():
            idx_fetch(c0 + 4, 0)

        idx_wait(c0 + 2, 2)
        gather_start(c0 + 2, 2, 0)
        consume(c0 + 1, 1, 1)

        @pl.when(not_last)
        def _():
            idx_fetch(c0 + 5, 1)

        idx_wait(c0 + 3, 3)
        gather_start(c0 + 3, 3, 1)
        consume(c0 + 2, 2, 0)

        @pl.when(not_last)
        def _():
            idx_fetch(c0 + 6, 2)

        @pl.when(not_last)
        def _():
            idx_wait(c0 + 4, 0)
            gather_start(c0 + 4, 0, 0)

        consume(c0 + 3, 3, 1)

        @pl.when(not_last)
        def _():
            idx_fetch(c0 + 7, 3)

        return 0

    lax.fori_loop(0, NCHUNK // 4, body, 0)
    plsc.subcore_barrier()
    pltpu.sync_copy(acc_s.at[pl.ds(sid * STRIPE, STRIPE)],
                    acc_out.at[cid, pl.ds(sid * STRIPE, STRIPE)])
    pltpu.sync_copy(s_s.at[pl.ds(sid * STRIPE, STRIPE)],
                    s_out.at[cid, pl.ds(sid * STRIPE, STRIPE)])


def _final_body(accp_ref, g_ref, st_ref, dinv_ref, b1_ref, w2_ref, b2_ref,
                out_ref, vacc_ref):
    i = pl.program_id(0)
    dinv = dinv_ref[...]
    s = st_ref[:, 0:1] + st_ref[:, 1:2]
    c = dinv * s + dinv * dinv
    ids = i * 128 + lax.broadcasted_iota(jnp.int32, (128, 1), 0)
    c = jnp.where(ids < N_NODES, c, 0.0)
    acc = accp_ref[0] + accp_ref[1] + g_ref[...]
    r = jnp.maximum(dinv * acc + b1_ref[...], 0.0)
    part = jnp.sum(c * r, axis=0, keepdims=True)

    @pl.when(i == 0)
    def _():
        vacc_ref[...] = part

    @pl.when(i > 0)
    def _():
        vacc_ref[...] += part

    @pl.when(i == NP // 128 - 1)
    def _():
        v = vacc_ref[...]
        out_ref[...] = (
            jax.lax.dot_general(
                v, w2_ref[...], (((1,), (0,)), ((), ())),
                preferred_element_type=jnp.float32,
                precision=lax.Precision.HIGHEST,
            ) / float(N_NODES) + b2_ref[...]
        )


def _run_final(accp, g, st, dinv, b1, W2, b2):
    return pl.pallas_call(
        _final_body,
        grid=(NP // 128,),
        in_specs=[
            pl.BlockSpec((2, 128, 128), lambda i: (0, i, 0)),
            pl.BlockSpec((128, 128), lambda i: (i, 0)),
            pl.BlockSpec((128, 2), lambda i: (i, 0)),
            pl.BlockSpec((128, 1), lambda i: (i, 0)),
            pl.BlockSpec((1, 128), lambda i: (0, 0)),
            pl.BlockSpec((128, 128), lambda i: (0, 0)),
            pl.BlockSpec((1, 128), lambda i: (0, 0)),
        ],
        out_specs=pl.BlockSpec((1, 128), lambda i: (0, 0)),
        out_shape=jax.ShapeDtypeStruct((1, 128), jnp.float32),
        scratch_shapes=[pltpu.VMEM((1, 128), jnp.float32)],
    )(accp, g, st, dinv, b1, W2, b2)


def kernel(x, edge_index, W1, b1, W2, b2):
    ei = edge_index.astype(jnp.int32)
    pad = N_NODES + (jnp.arange(EP - E, dtype=jnp.int32) % N_PAD_ROWS)
    src = jnp.concatenate([ei[0], pad])
    dst = jnp.concatenate([ei[1], pad])
    src3 = src.reshape(NW, NCHUNK, CH)
    dst3 = dst.reshape(NW, NCHUNK, CH)
    si = jnp.stack([src3, dst3], axis=2)
    x_p = jnp.concatenate([x, jnp.zeros((NP - N_NODES, 128), x.dtype)])

    degp = _deg_kernel(dst3)
    degt = degp.T.reshape(NP, 2)
    g, dinv = _run_scale(x_p, degt, W1)
    accp, sp = _agg_kernel(si, g, dinv.reshape(NP))
    st = sp.T.reshape(NP, 2)
    out = _run_final(accp, g, st, dinv, b1.reshape(1, 128), W2,
                     b2.reshape(1, 128))
    return out.reshape(128)


# speedup vs baseline: 51.7912x; 1.3860x over previous
import functools

import jax
import jax.numpy as jnp
from jax import lax
from jax.experimental import pallas as pl
from jax.experimental.pallas import tpu as pltpu
from jax.experimental.pallas import tpu_sc as plsc

N_NODES = 10000
NP = 10240
N_PAD_ROWS = NP - N_NODES
E = 320000
EP = 327680
NW = 32
CH = 128
NCHUNK = 80
EPW = NCHUNK * CH
STRIPE = NP // 16

_mesh = plsc.VectorSubcoreMesh(core_axis_name="c", subcore_axis_name="s")


def _zero_1d(ref, n):
    z = jnp.zeros((16,), jnp.float32)

    def body(i, _):
        ref[pl.ds(i * 16, 16)] = z
        return 0

    lax.fori_loop(0, n // 16, body, 0)


def _zero_2d(ref, rows):
    z = jnp.zeros((16,), jnp.float32)

    def body(i, _):
        for k in range(8):
            ref[i, pl.ds(k * 16, 16)] = z
        return 0

    lax.fori_loop(0, rows, body, 0)


@functools.partial(
    pl.kernel,
    out_type=jax.ShapeDtypeStruct((2, NP), jnp.float32),
    mesh=_mesh,
    scratch_types=[
        pltpu.VMEM((NCHUNK, CH), jnp.int32),
        pltpu.VMEM((CH,), jnp.float32),
        pltpu.VMEM((STRIPE,), jnp.float32),
        pltpu.VMEM_SHARED((NP,), jnp.float32),
    ],
    compiler_params=pltpu.CompilerParams(needs_layout_passes=False),
)
def _deg_kernel(dst_hbm, deg_out, dst_v, ones_v, z_v, deg_s):
    cid = lax.axis_index("c")
    sid = lax.axis_index("s")
    wid = sid * 2 + cid
    pltpu.sync_copy(dst_hbm.at[wid], dst_v)
    one = jnp.ones((16,), jnp.float32)
    for k in range(CH // 16):
        ones_v[pl.ds(k * 16, 16)] = one
    _zero_1d(z_v, STRIPE)
    pltpu.sync_copy(z_v, deg_s.at[pl.ds(sid * STRIPE, STRIPE)])
    plsc.subcore_barrier()

    def body(j, _):
        pltpu.sync_copy(ones_v, deg_s.at[dst_v.at[j]], add=True)
        return 0

    lax.fori_loop(0, NCHUNK, body, 0)
    plsc.subcore_barrier()
    pltpu.sync_copy(deg_s.at[pl.ds(sid * STRIPE, STRIPE)],
                    deg_out.at[cid, pl.ds(sid * STRIPE, STRIPE)])


RB = 1024


def _matmul_body(x_ref, w1_ref, h_ref):
    h_ref[...] = jax.lax.dot_general(
        x_ref[...], w1_ref[...], (((1,), (0,)), ((), ())),
        preferred_element_type=jnp.float32,
        precision=lax.Precision.HIGHEST,
    )


def _run_matmul(x_p, W1):
    return pl.pallas_call(
        _matmul_body,
        grid=(NP // RB,),
        in_specs=[
            pl.BlockSpec((RB, 128), lambda i: (i, 0)),
            pl.BlockSpec((128, 128), lambda i: (0, 0)),
        ],
        out_specs=pl.BlockSpec((RB, 128), lambda i: (i, 0)),
        out_shape=jax.ShapeDtypeStruct((NP, 128), jnp.float32),
    )(x_p, W1)


def _scale_body(h_ref, degt_ref, g_ref, dinv_ref):
    deg = degt_ref[:, 0:1] + degt_ref[:, 1:2] + 1.0
    dinv = lax.rsqrt(deg)
    g_ref[...] = dinv * h_ref[...]
    dinv_ref[...] = dinv


def _run_scale(h, degt):
    return pl.pallas_call(
        _scale_body,
        grid=(NP // RB,),
        in_specs=[
            pl.BlockSpec((RB, 128), lambda i: (i, 0)),
            pl.BlockSpec((RB, 2), lambda i: (i, 0)),
        ],
        out_specs=[
            pl.BlockSpec((RB, 128), lambda i: (i, 0)),
            pl.BlockSpec((RB, 1), lambda i: (i, 0)),
        ],
        out_shape=[
            jax.ShapeDtypeStruct((NP, 128), jnp.float32),
            jax.ShapeDtypeStruct((NP, 1), jnp.float32),
        ],
    )(h, degt)


@functools.partial(
    pl.kernel,
    out_type=(
        jax.ShapeDtypeStruct((2, NP, 128), jnp.float32),
        jax.ShapeDtypeStruct((2, NP), jnp.float32),
    ),
    mesh=_mesh,
    scratch_types=[
        pltpu.VMEM((4, CH), jnp.int32),
        pltpu.VMEM((4, CH), jnp.int32),
        pltpu.VMEM((NP,), jnp.float32),
        pltpu.VMEM((CH,), jnp.float32),
        pltpu.VMEM((CH, 128), jnp.float32),
        pltpu.VMEM((CH, 128), jnp.float32),
        pltpu.VMEM_SHARED((NP, 128), jnp.float32),
        pltpu.VMEM_SHARED((NP,), jnp.float32),
        pltpu.SemaphoreType.DMA,
        pltpu.SemaphoreType.DMA,
        pltpu.SemaphoreType.DMA,
        pltpu.SemaphoreType.DMA,
        pltpu.SemaphoreType.DMA,
        pltpu.SemaphoreType.DMA,
        pltpu.SemaphoreType.DMA,
        pltpu.SemaphoreType.DMA,
        pltpu.SemaphoreType.DMA,
        pltpu.SemaphoreType.DMA,
    ],
    compiler_params=pltpu.CompilerParams(needs_layout_passes=False),
)
def _agg_kernel(src_hbm, dst_hbm, g_hbm, dinv_hbm,
                acc_out, s_out,
                sslot_v, dslot_v, dinv_v, sval_v, rows0_v, rows1_v,
                acc_s, s_s,
                isem0, isem1, isem2, isem3,
                jsem0, jsem1, jsem2, jsem3, gsem0, gsem1):
    cid = lax.axis_index("c")
    sid = lax.axis_index("s")
    wid = sid * 2 + cid
    isems = (isem0, isem1, isem2, isem3)
    jsems = (jsem0, jsem1, jsem2, jsem3)
    rows = (rows0_v, rows1_v)
    gsems = (gsem0, gsem1)
    pltpu.sync_copy(dinv_hbm, dinv_v)
    _zero_2d(rows0_v, CH)
    for t in range(STRIPE // CH):
        pltpu.sync_copy(rows0_v, acc_s.at[pl.ds(sid * STRIPE + t * CH, CH)])
    _zero_1d(sval_v, CH)
    for t in range(STRIPE // CH):
        pltpu.sync_copy(sval_v, s_s.at[pl.ds(sid * STRIPE + t * CH, CH)])
    plsc.subcore_barrier()

    def idx_fetch(c, slot):
        pltpu.async_copy(src_hbm.at[wid, c], sslot_v.at[slot], isems[slot])
        pltpu.async_copy(dst_hbm.at[wid, c], dslot_v.at[slot], jsems[slot])

    def idx_wait(c, slot):
        pltpu.make_async_copy(src_hbm.at[wid, c], sslot_v.at[slot],
                              isems[slot]).wait()
        pltpu.make_async_copy(dst_hbm.at[wid, c], dslot_v.at[slot],
                              jsems[slot]).wait()

    def gather_start(c, slot, buf):
        pltpu.async_copy(g_hbm.at[sslot_v.at[slot]], rows[buf], gsems[buf])

    def gather_wait(c, slot, buf):
        pltpu.make_async_copy(g_hbm.at[sslot_v.at[slot]], rows[buf],
                              gsems[buf]).wait()

    def s_part(slot):
        for k in range(CH // 16):
            di = dslot_v[slot, pl.ds(k * 16, 16)]
            sval_v[pl.ds(k * 16, 16)] = plsc.load_gather(dinv_v, [di])

    def consume(c, slot, buf):
        s_part(slot)
        gather_wait(c, slot, buf)
        pltpu.sync_copy(rows[buf], acc_s.at[dslot_v.at[slot]], add=True)
        pltpu.sync_copy(sval_v, s_s.at[sslot_v.at[slot]], add=True)

    for c in range(4):
        idx_fetch(c, c)
    idx_wait(0, 0)
    gather_start(0, 0, 0)

    def body(t, _):
        c0 = 4 * t
        not_last = t < NCHUNK // 4 - 1
        idx_wait(c0 + 1, 1)
        gather_start(c0 + 1, 1, 1)
        consume(c0, 0, 0)

        @pl.when(not_last)
        def _():
            idx_fetch(c0 + 4, 0)

        idx_wait(c0 + 2, 2)
        gather_start(c0 + 2, 2, 0)
        consume(c0 + 1, 1, 1)

        @pl.when(not_last)
        def _():
            idx_fetch(c0 + 5, 1)

        idx_wait(c0 + 3, 3)
        gather_start(c0 + 3, 3, 1)
        consume(c0 + 2, 2, 0)

        @pl.when(not_last)
        def _():
            idx_fetch(c0 + 6, 2)

        @pl.when(not_last)
        def _():
            idx_wait(c0 + 4, 0)
            gather_start(c0 + 4, 0, 0)

        consume(c0 + 3, 3, 1)

        @pl.when(not_last)
        def _():
            idx_fetch(c0 + 7, 3)

        return 0

    lax.fori_loop(0, NCHUNK // 4, body, 0)
    plsc.subcore_barrier()
    pltpu.sync_copy(acc_s.at[pl.ds(sid * STRIPE, STRIPE)],
                    acc_out.at[cid, pl.ds(sid * STRIPE, STRIPE)])
    pltpu.sync_copy(s_s.at[pl.ds(sid * STRIPE, STRIPE)],
                    s_out.at[cid, pl.ds(sid * STRIPE, STRIPE)])


def _final_body(accp_ref, g_ref, st_ref, dinv_ref, b1_ref, w2_ref, b2_ref,
                out_ref, vacc_ref):
    i = pl.program_id(0)
    dinv = dinv_ref[...]
    s = st_ref[:, 0:1] + st_ref[:, 1:2]
    c = dinv * s + dinv * dinv
    ids = i * RB + lax.broadcasted_iota(jnp.int32, (RB, 1), 0)
    c = jnp.where(ids < N_NODES, c, 0.0)
    acc = accp_ref[0] + accp_ref[1] + g_ref[...]
    r = jnp.maximum(dinv * acc + b1_ref[...], 0.0)
    part = jnp.sum(c * r, axis=0, keepdims=True)

    @pl.when(i == 0)
    def _():
        vacc_ref[...] = part

    @pl.when(i > 0)
    def _():
        vacc_ref[...] += part

    @pl.when(i == NP // RB - 1)
    def _():
        v = vacc_ref[...]
        out_ref[...] = (
            jax.lax.dot_general(
                v, w2_ref[...], (((1,), (0,)), ((), ())),
                preferred_element_type=jnp.float32,
                precision=lax.Precision.HIGHEST,
            ) / float(N_NODES) + b2_ref[...]
        )


def _run_final(accp, g, st, dinv, b1, W2, b2):
    return pl.pallas_call(
        _final_body,
        grid=(NP // RB,),
        in_specs=[
            pl.BlockSpec((2, RB, 128), lambda i: (0, i, 0)),
            pl.BlockSpec((RB, 128), lambda i: (i, 0)),
            pl.BlockSpec((RB, 2), lambda i: (i, 0)),
            pl.BlockSpec((RB, 1), lambda i: (i, 0)),
            pl.BlockSpec((1, 128), lambda i: (0, 0)),
            pl.BlockSpec((128, 128), lambda i: (0, 0)),
            pl.BlockSpec((1, 128), lambda i: (0, 0)),
        ],
        out_specs=pl.BlockSpec((1, 128), lambda i: (0, 0)),
        out_shape=jax.ShapeDtypeStruct((1, 128), jnp.float32),
        scratch_shapes=[pltpu.VMEM((1, 128), jnp.float32)],
    )(accp, g, st, dinv, b1, W2, b2)


def kernel(x, edge_index, W1, b1, W2, b2):
    ei = edge_index.astype(jnp.int32)
    pad = N_NODES + (jnp.arange(EP - E, dtype=jnp.int32) % N_PAD_ROWS)
    src = jnp.concatenate([ei[0], pad])
    dst = jnp.concatenate([ei[1], pad])
    src3 = src.reshape(NW, NCHUNK, CH)
    dst3 = dst.reshape(NW, NCHUNK, CH)
    x_p = jnp.concatenate([x, jnp.zeros((NP - N_NODES, 128), x.dtype)])

    h = _run_matmul(x_p, W1)
    degp = _deg_kernel(dst3)
    degt = degp.T.reshape(NP, 2)
    g, dinv = _run_scale(h, degt)
    accp, sp = _agg_kernel(src3, dst3, g, dinv.reshape(NP))
    st = sp.T.reshape(NP, 2)
    out = _run_final(accp, g, st, dinv, b1.reshape(1, 128), W2,
                     b2.reshape(1, 128))
    return out.reshape(128)


# speedup vs baseline: 52.7509x; 1.0185x over previous
import functools

import jax
import jax.numpy as jnp
from jax import lax
from jax.experimental import pallas as pl
from jax.experimental.pallas import tpu as pltpu
from jax.experimental.pallas import tpu_sc as plsc

N_NODES = 10000
NP = 10240
N_PAD_ROWS = NP - N_NODES
E = 320000
EP = 327680
NW = 32
CH = 128
NCHUNK = 80
EPW = NCHUNK * CH
STRIPE = NP // 16

_mesh = plsc.VectorSubcoreMesh(core_axis_name="c", subcore_axis_name="s")


def _zero_1d(ref, n):
    z = jnp.zeros((16,), jnp.float32)

    def body(i, _):
        ref[pl.ds(i * 16, 16)] = z
        return 0

    lax.fori_loop(0, n // 16, body, 0)


def _zero_2d(ref, rows):
    z = jnp.zeros((16,), jnp.float32)

    def body(i, _):
        for k in range(8):
            ref[i, pl.ds(k * 16, 16)] = z
        return 0

    lax.fori_loop(0, rows, body, 0)


@functools.partial(
    pl.kernel,
    out_type=jax.ShapeDtypeStruct((2, NP), jnp.float32),
    mesh=_mesh,
    scratch_types=[
        pltpu.VMEM((4, 2, CH), jnp.int32),
        pltpu.VMEM((CH,), jnp.float32),
        pltpu.VMEM((STRIPE,), jnp.float32),
        pltpu.VMEM_SHARED((NP,), jnp.float32),
        pltpu.SemaphoreType.DMA,
        pltpu.SemaphoreType.DMA,
        pltpu.SemaphoreType.DMA,
        pltpu.SemaphoreType.DMA,
        pltpu.SemaphoreType.DMA,
        pltpu.SemaphoreType.DMA,
    ],
    compiler_params=pltpu.CompilerParams(needs_layout_passes=False),
)
def _deg_kernel(ei_hbm, deg_out, sd_v, ones_v, z_v, deg_s,
                isem0, isem1, isem2, isem3, dsem0, dsem1):
    cid = lax.axis_index("c")
    sid = lax.axis_index("s")
    wid = sid * 2 + cid
    isems = (isem0, isem1, isem2, isem3)
    dsems = (dsem0, dsem1)
    one = jnp.ones((16,), jnp.float32)
    for k in range(CH // 16):
        ones_v[pl.ds(k * 16, 16)] = one
    _zero_1d(z_v, STRIPE)
    pltpu.sync_copy(z_v, deg_s.at[pl.ds(sid * STRIPE, STRIPE)])
    plsc.subcore_barrier()

    def idx_fetch(c, slot):
        pltpu.async_copy(ei_hbm.at[:, pl.ds(wid * EPW + c * CH, CH)],
                         sd_v.at[slot], isems[slot])

    def idx_wait(c, slot):
        pltpu.make_async_copy(ei_hbm.at[:, pl.ds(wid * EPW + c * CH, CH)],
                              sd_v.at[slot], isems[slot]).wait()

    def scat_wait(slot, buf):
        pltpu.make_async_copy(ones_v, deg_s.at[sd_v.at[slot, 1]],
                              dsems[buf]).wait()

    def consume(c, slot, buf):
        @pl.when(c >= 2)
        def _():
            scat_wait((slot + 2) % 4, buf)

        @pl.when(c + 2 < NCHUNK)
        def _():
            idx_fetch(c + 2, (slot + 2) % 4)

        idx_wait(c, slot)
        pltpu.async_copy(ones_v, deg_s.at[sd_v.at[slot, 1]], dsems[buf],
                         add=True)

    idx_fetch(0, 0)
    idx_fetch(1, 1)

    def body(t, _):
        c0 = 4 * t
        consume(c0, 0, 0)
        consume(c0 + 1, 1, 1)
        consume(c0 + 2, 2, 0)
        consume(c0 + 3, 3, 1)
        return 0

    lax.fori_loop(0, NCHUNK // 4, body, 0)
    scat_wait((NCHUNK - 2) % 4, 0)
    scat_wait((NCHUNK - 1) % 4, 1)
    plsc.subcore_barrier()
    pltpu.sync_copy(deg_s.at[pl.ds(sid * STRIPE, STRIPE)],
                    deg_out.at[cid, pl.ds(sid * STRIPE, STRIPE)])


RB = 1024


def _matmul_body(x_ref, w1_ref, h_ref):
    h_ref[...] = jax.lax.dot_general(
        x_ref[...], w1_ref[...], (((1,), (0,)), ((), ())),
        preferred_element_type=jnp.float32,
        precision=lax.Precision.HIGHEST,
    )


def _run_matmul(x_p, W1):
    return pl.pallas_call(
        _matmul_body,
        grid=(NP // RB,),
        in_specs=[
            pl.BlockSpec((RB, 128), lambda i: (i, 0)),
            pl.BlockSpec((128, 128), lambda i: (0, 0)),
        ],
        out_specs=pl.BlockSpec((RB, 128), lambda i: (i, 0)),
        out_shape=jax.ShapeDtypeStruct((NP, 128), jnp.float32),
    )(x_p, W1)


def _scale_body(h_ref, degt_ref, g_ref, dinv_ref):
    deg = degt_ref[:, 0:1] + degt_ref[:, 1:2] + 1.0
    dinv = lax.rsqrt(deg)
    g_ref[...] = dinv * h_ref[...]
    dinv_ref[...] = dinv


def _run_scale(h, degt):
    return pl.pallas_call(
        _scale_body,
        grid=(NP // RB,),
        in_specs=[
            pl.BlockSpec((RB, 128), lambda i: (i, 0)),
            pl.BlockSpec((RB, 2), lambda i: (i, 0)),
        ],
        out_specs=[
            pl.BlockSpec((RB, 128), lambda i: (i, 0)),
            pl.BlockSpec((RB, 1), lambda i: (i, 0)),
        ],
        out_shape=[
            jax.ShapeDtypeStruct((NP, 128), jnp.float32),
            jax.ShapeDtypeStruct((NP, 1), jnp.float32),
        ],
    )(h, degt)


@functools.partial(
    pl.kernel,
    out_type=(
        jax.ShapeDtypeStruct((2, NP, 128), jnp.float32),
        jax.ShapeDtypeStruct((2, NP), jnp.float32),
    ),
    mesh=_mesh,
    scratch_types=[
        pltpu.VMEM((4, 2, CH), jnp.int32),
        pltpu.VMEM((NP,), jnp.float32),
        pltpu.VMEM((CH,), jnp.float32),
        pltpu.VMEM((CH,), jnp.float32),
        pltpu.VMEM((CH, 128), jnp.float32),
        pltpu.VMEM((CH, 128), jnp.float32),
        pltpu.VMEM_SHARED((NP, 128), jnp.float32),
        pltpu.VMEM_SHARED((NP,), jnp.float32),
        pltpu.SemaphoreType.DMA,
        pltpu.SemaphoreType.DMA,
        pltpu.SemaphoreType.DMA,
        pltpu.SemaphoreType.DMA,
        pltpu.SemaphoreType.DMA,
        pltpu.SemaphoreType.DMA,
        pltpu.SemaphoreType.DMA,
        pltpu.SemaphoreType.DMA,
        pltpu.SemaphoreType.DMA,
        pltpu.SemaphoreType.DMA,
    ],
    compiler_params=pltpu.CompilerParams(needs_layout_passes=False),
)
def _agg_kernel(ei_hbm, g_hbm, dinv_hbm,
                acc_out, s_out,
                sd_v, dinv_v, sval0_v, sval1_v, rows0_v, rows1_v,
                acc_s, s_s,
                isem0, isem1, isem2, isem3,
                gsem0, gsem1, ssem0, ssem1, tsem0, tsem1):
    cid = lax.axis_index("c")
    sid = lax.axis_index("s")
    wid = sid * 2 + cid
    isems = (isem0, isem1, isem2, isem3)
    rows = (rows0_v, rows1_v)
    svals = (sval0_v, sval1_v)
    gsems = (gsem0, gsem1)
    ssems = (ssem0, ssem1)
    tsems = (tsem0, tsem1)
    pltpu.sync_copy(dinv_hbm, dinv_v)
    _zero_2d(rows0_v, CH)
    for t in range(STRIPE // CH):
        pltpu.sync_copy(rows0_v, acc_s.at[pl.ds(sid * STRIPE + t * CH, CH)])
    _zero_1d(sval0_v, CH)
    for t in range(STRIPE // CH):
        pltpu.sync_copy(sval0_v, s_s.at[pl.ds(sid * STRIPE + t * CH, CH)])
    plsc.subcore_barrier()

    def idx_fetch(c, slot):
        pltpu.async_copy(ei_hbm.at[:, pl.ds(wid * EPW + c * CH, CH)],
                         sd_v.at[slot], isems[slot])

    def idx_wait(c, slot):
        pltpu.make_async_copy(ei_hbm.at[:, pl.ds(wid * EPW + c * CH, CH)],
                              sd_v.at[slot], isems[slot]).wait()

    def scat_wait(slot, buf):
        pltpu.make_async_copy(rows[buf], acc_s.at[sd_v.at[slot, 1]],
                              ssems[buf]).wait()

    def t_wait(slot, buf):
        pltpu.make_async_copy(svals[buf], s_s.at[sd_v.at[slot, 0]],
                              tsems[buf]).wait()

    def gather_start(c, slot, buf):
        @pl.when(c >= 2)
        def _():
            scat_wait((slot + 2) % 4, buf)

        pltpu.async_copy(g_hbm.at[sd_v.at[slot, 0]], rows[buf], gsems[buf])

    def gather_wait(c, slot, buf):
        pltpu.make_async_copy(g_hbm.at[sd_v.at[slot, 0]], rows[buf],
                              gsems[buf]).wait()

    def s_part(slot, buf):
        for k in range(CH // 16):
            di = sd_v[slot, 1, pl.ds(k * 16, 16)]
            svals[buf][pl.ds(k * 16, 16)] = plsc.load_gather(dinv_v, [di])

    def consume(c, slot, buf):
        @pl.when(c >= 2)
        def _():
            t_wait((slot + 2) % 4, buf)

        @pl.when(c + 2 < NCHUNK)
        def _():
            idx_fetch(c + 2, (slot + 2) % 4)

        s_part(slot, buf)
        gather_wait(c, slot, buf)
        pltpu.async_copy(rows[buf], acc_s.at[sd_v.at[slot, 1]], ssems[buf],
                         add=True)
        pltpu.async_copy(svals[buf], s_s.at[sd_v.at[slot, 0]], tsems[buf],
                         add=True)

    idx_fetch(0, 0)
    idx_fetch(1, 1)
    idx_wait(0, 0)
    gather_start(0, 0, 0)

    def body(t, _):
        c0 = 4 * t
        idx_wait(c0 + 1, 1)
        gather_start(c0 + 1, 1, 1)
        consume(c0, 0, 0)
        idx_wait(c0 + 2, 2)
        gather_start(c0 + 2, 2, 0)
        consume(c0 + 1, 1, 1)
        idx_wait(c0 + 3, 3)
        gather_start(c0 + 3, 3, 1)
        consume(c0 + 2, 2, 0)

        @pl.when(t < NCHUNK // 4 - 1)
        def _():
            idx_wait(c0 + 4, 0)
            gather_start(c0 + 4, 0, 0)

        consume(c0 + 3, 3, 1)
        return 0

    lax.fori_loop(0, NCHUNK // 4, body, 0)
    scat_wait((NCHUNK - 2) % 4, 0)
    scat_wait((NCHUNK - 1) % 4, 1)
    t_wait((NCHUNK - 2) % 4, 0)
    t_wait((NCHUNK - 1) % 4, 1)
    plsc.subcore_barrier()
    pltpu.sync_copy(acc_s.at[pl.ds(sid * STRIPE, STRIPE)],
                    acc_out.at[cid, pl.ds(sid * STRIPE, STRIPE)])
    pltpu.sync_copy(s_s.at[pl.ds(sid * STRIPE, STRIPE)],
                    s_out.at[cid, pl.ds(sid * STRIPE, STRIPE)])


def _final_body(accp_ref, g_ref, st_ref, dinv_ref, b1_ref, w2_ref, b2_ref,
                out_ref, vacc_ref):
    i = pl.program_id(0)
    dinv = dinv_ref[...]
    s = st_ref[:, 0:1] + st_ref[:, 1:2]
    c = dinv * s + dinv * dinv
    ids = i * RB + lax.broadcasted_iota(jnp.int32, (RB, 1), 0)
    c = jnp.where(ids < N_NODES, c, 0.0)
    acc = accp_ref[0] + accp_ref[1] + g_ref[...]
    r = jnp.maximum(dinv * acc + b1_ref[...], 0.0)
    part = jnp.sum(c * r, axis=0, keepdims=True)

    @pl.when(i == 0)
    def _():
        vacc_ref[...] = part

    @pl.when(i > 0)
    def _():
        vacc_ref[...] += part

    @pl.when(i == NP // RB - 1)
    def _():
        v = vacc_ref[...]
        out_ref[...] = (
            jax.lax.dot_general(
                v, w2_ref[...], (((1,), (0,)), ((), ())),
                preferred_element_type=jnp.float32,
                precision=lax.Precision.HIGHEST,
            ) / float(N_NODES) + b2_ref[...]
        )


def _run_final(accp, g, st, dinv, b1, W2, b2):
    return pl.pallas_call(
        _final_body,
        grid=(NP // RB,),
        in_specs=[
            pl.BlockSpec((2, RB, 128), lambda i: (0, i, 0)),
            pl.BlockSpec((RB, 128), lambda i: (i, 0)),
            pl.BlockSpec((RB, 2), lambda i: (i, 0)),
            pl.BlockSpec((RB, 1), lambda i: (i, 0)),
            pl.BlockSpec((1, 128), lambda i: (0, 0)),
            pl.BlockSpec((128, 128), lambda i: (0, 0)),
            pl.BlockSpec((1, 128), lambda i: (0, 0)),
        ],
        out_specs=pl.BlockSpec((1, 128), lambda i: (0, 0)),
        out_shape=jax.ShapeDtypeStruct((1, 128), jnp.float32),
        scratch_shapes=[pltpu.VMEM((1, 128), jnp.float32)],
    )(accp, g, st, dinv, b1, W2, b2)


def kernel(x, edge_index, W1, b1, W2, b2):
    ei = edge_index.astype(jnp.int32)
    pad = N_NODES + (jnp.arange(EP - E, dtype=jnp.int32) % N_PAD_ROWS)
    ei2 = jnp.concatenate(
        [ei, jnp.broadcast_to(pad[None, :], (2, EP - E))], axis=1)
    x_p = jnp.concatenate([x, jnp.zeros((NP - N_NODES, 128), x.dtype)])

    h = _run_matmul(x_p, W1)
    degp = _deg_kernel(ei2)
    degt = degp.T.reshape(NP, 2)
    g, dinv = _run_scale(h, degt)
    accp, sp = _agg_kernel(ei2, g, dinv.reshape(NP))
    st = sp.T.reshape(NP, 2)
    out = _run_final(accp, g, st, dinv, b1.reshape(1, 128), W2,
                     b2.reshape(1, 128))
    return out.reshape(128)


# speedup vs baseline: 53.7318x; 1.0186x over previous
import functools

import jax
import jax.numpy as jnp
from jax import lax
from jax.experimental import pallas as pl
from jax.experimental.pallas import tpu as pltpu
from jax.experimental.pallas import tpu_sc as plsc

N_NODES = 10000
NP = 10240
N_PAD_ROWS = NP - N_NODES
E = 320000
EP = 327680
NW = 32
CH = 128
NCHUNK = 80
EPW = NCHUNK * CH
STRIPE = NP // 16

_mesh = plsc.VectorSubcoreMesh(core_axis_name="c", subcore_axis_name="s")


def _zero_1d(ref, n):
    z = jnp.zeros((16,), jnp.float32)

    def body(i, _):
        ref[pl.ds(i * 16, 16)] = z
        return 0

    lax.fori_loop(0, n // 16, body, 0)


def _zero_2d(ref, rows):
    z = jnp.zeros((16,), jnp.float32)

    def body(i, _):
        for k in range(8):
            ref[i, pl.ds(k * 16, 16)] = z
        return 0

    lax.fori_loop(0, rows, body, 0)


@functools.partial(
    pl.kernel,
    out_type=jax.ShapeDtypeStruct((2, NP // 128, 128), jnp.float32),
    mesh=_mesh,
    scratch_types=[
        pltpu.VMEM((EPW,), jnp.int32),
        pltpu.VMEM((NP // 128, 128), jnp.float32),
        pltpu.VMEM((NP // 128,), jnp.int32),
        pltpu.VMEM_SHARED((NP // 128, 128), jnp.float32),
    ],
    compiler_params=pltpu.CompilerParams(needs_layout_passes=False),
)
def _deg_kernel(ei_hbm, deg_out, dst_v, hist_v, iota_v, deg_s):
    cid = lax.axis_index("c")
    sid = lax.axis_index("s")
    wid = sid * 2 + cid
    NR = NP // 128
    pltpu.sync_copy(ei_hbm.at[1, pl.ds(wid * EPW, EPW)], dst_v)
    _zero_2d(hist_v, NR)
    lane = lax.iota(jnp.int32, 16)
    for k in range(NR // 16):
        iota_v[pl.ds(k * 16, 16)] = lane + k * 16
    @pl.when(sid < NR // 8)
    def _():
        pltpu.sync_copy(hist_v.at[pl.ds(sid * 8, 8)],
                        deg_s.at[pl.ds(sid * 8, 8)])

    one = jnp.ones((16,), jnp.float32)

    def body(j, _):
        di = dst_v[pl.ds(j * 16, 16)]
        plsc.addupdate_scatter(hist_v, [di >> 7, di & 127], one)
        return 0

    lax.fori_loop(0, EPW // 16, body, 0)
    plsc.subcore_barrier()
    pltpu.sync_copy(hist_v, deg_s.at[iota_v], add=True)
    plsc.subcore_barrier()

    @pl.when(sid < NR // 8)
    def _():
        pltpu.sync_copy(deg_s.at[pl.ds(sid * 8, 8)],
                        deg_out.at[cid, pl.ds(sid * 8, 8)])


RB = 1024


def _matmul_body(x_ref, w1_ref, h_ref):
    h_ref[...] = jax.lax.dot_general(
        x_ref[...], w1_ref[...], (((1,), (0,)), ((), ())),
        preferred_element_type=jnp.float32,
        precision=lax.Precision.HIGHEST,
    )


def _run_matmul(x_p, W1):
    return pl.pallas_call(
        _matmul_body,
        grid=(NP // RB,),
        in_specs=[
            pl.BlockSpec((RB, 128), lambda i: (i, 0)),
            pl.BlockSpec((128, 128), lambda i: (0, 0)),
        ],
        out_specs=pl.BlockSpec((RB, 128), lambda i: (i, 0)),
        out_shape=jax.ShapeDtypeStruct((NP, 128), jnp.float32),
    )(x_p, W1)


def _scale_body(h_ref, degt_ref, g_ref, dinv_ref):
    deg = degt_ref[:, 0:1] + degt_ref[:, 1:2] + 1.0
    dinv = lax.rsqrt(deg)
    g_ref[...] = dinv * h_ref[...]
    dinv_ref[...] = dinv


def _run_scale(h, degt):
    return pl.pallas_call(
        _scale_body,
        grid=(NP // RB,),
        in_specs=[
            pl.BlockSpec((RB, 128), lambda i: (i, 0)),
            pl.BlockSpec((RB, 2), lambda i: (i, 0)),
        ],
        out_specs=[
            pl.BlockSpec((RB, 128), lambda i: (i, 0)),
            pl.BlockSpec((RB, 1), lambda i: (i, 0)),
        ],
        out_shape=[
            jax.ShapeDtypeStruct((NP, 128), jnp.float32),
            jax.ShapeDtypeStruct((NP, 1), jnp.float32),
        ],
    )(h, degt)


@functools.partial(
    pl.kernel,
    out_type=(
        jax.ShapeDtypeStruct((2, NP, 128), jnp.float32),
        jax.ShapeDtypeStruct((2, NP), jnp.float32),
    ),
    mesh=_mesh,
    scratch_types=[
        pltpu.VMEM((4, 2, CH), jnp.int32),
        pltpu.VMEM((NP,), jnp.float32),
        pltpu.VMEM((CH,), jnp.float32),
        pltpu.VMEM((CH,), jnp.float32),
        pltpu.VMEM((CH, 128), jnp.float32),
        pltpu.VMEM((CH, 128), jnp.float32),
        pltpu.VMEM_SHARED((NP, 128), jnp.float32),
        pltpu.VMEM_SHARED((NP,), jnp.float32),
        pltpu.SemaphoreType.DMA,
        pltpu.SemaphoreType.DMA,
        pltpu.SemaphoreType.DMA,
        pltpu.SemaphoreType.DMA,
        pltpu.SemaphoreType.DMA,
        pltpu.SemaphoreType.DMA,
        pltpu.SemaphoreType.DMA,
        pltpu.SemaphoreType.DMA,
        pltpu.SemaphoreType.DMA,
        pltpu.SemaphoreType.DMA,
    ],
    compiler_params=pltpu.CompilerParams(needs_layout_passes=False),
)
def _agg_kernel(ei_hbm, g_hbm, dinv_hbm,
                acc_out, s_out,
                sd_v, dinv_v, sval0_v, sval1_v, rows0_v, rows1_v,
                acc_s, s_s,
                isem0, isem1, isem2, isem3,
                gsem0, gsem1, ssem0, ssem1, tsem0, tsem1):
    cid = lax.axis_index("c")
    sid = lax.axis_index("s")
    wid = sid * 2 + cid
    isems = (isem0, isem1, isem2, isem3)
    rows = (rows0_v, rows1_v)
    svals = (sval0_v, sval1_v)
    gsems = (gsem0, gsem1)
    ssems = (ssem0, ssem1)
    tsems = (tsem0, tsem1)
    pltpu.sync_copy(dinv_hbm, dinv_v)
    _zero_2d(rows0_v, CH)
    for t in range(STRIPE // CH):
        pltpu.sync_copy(rows0_v, acc_s.at[pl.ds(sid * STRIPE + t * CH, CH)])
    _zero_1d(sval0_v, CH)
    for t in range(STRIPE // CH):
        pltpu.sync_copy(sval0_v, s_s.at[pl.ds(sid * STRIPE + t * CH, CH)])
    plsc.subcore_barrier()

    def idx_fetch(c, slot):
        pltpu.async_copy(ei_hbm.at[:, pl.ds(wid * EPW + c * CH, CH)],
                         sd_v.at[slot], isems[slot])

    def idx_wait(c, slot):
        pltpu.make_async_copy(ei_hbm.at[:, pl.ds(wid * EPW + c * CH, CH)],
                              sd_v.at[slot], isems[slot]).wait()

    def scat_wait(slot, buf):
        pltpu.make_async_copy(rows[buf], acc_s.at[sd_v.at[slot, 1]],
                              ssems[buf]).wait()

    def t_wait(slot, buf):
        pltpu.make_async_copy(svals[buf], s_s.at[sd_v.at[slot, 0]],
                              tsems[buf]).wait()

    def gather_start(c, slot, buf):
        @pl.when(c >= 2)
        def _():
            scat_wait((slot + 2) % 4, buf)

        pltpu.async_copy(g_hbm.at[sd_v.at[slot, 0]], rows[buf], gsems[buf])

    def gather_wait(c, slot, buf):
        pltpu.make_async_copy(g_hbm.at[sd_v.at[slot, 0]], rows[buf],
                              gsems[buf]).wait()

    def s_part(slot, buf):
        for k in range(CH // 16):
            di = sd_v[slot, 1, pl.ds(k * 16, 16)]
            svals[buf][pl.ds(k * 16, 16)] = plsc.load_gather(dinv_v, [di])

    def consume(c, slot, buf):
        @pl.when(c >= 2)
        def _():
            t_wait((slot + 2) % 4, buf)

        @pl.when(c + 2 < NCHUNK)
        def _():
            idx_fetch(c + 2, (slot + 2) % 4)

        s_part(slot, buf)
        gather_wait(c, slot, buf)
        pltpu.async_copy(rows[buf], acc_s.at[sd_v.at[slot, 1]], ssems[buf],
                         add=True)
        pltpu.async_copy(svals[buf], s_s.at[sd_v.at[slot, 0]], tsems[buf],
                         add=True)

    idx_fetch(0, 0)
    idx_fetch(1, 1)
    idx_wait(0, 0)
    gather_start(0, 0, 0)

    def body(t, _):
        c0 = 4 * t
        idx_wait(c0 + 1, 1)
        gather_start(c0 + 1, 1, 1)
        consume(c0, 0, 0)
        idx_wait(c0 + 2, 2)
        gather_start(c0 + 2, 2, 0)
        consume(c0 + 1, 1, 1)
        idx_wait(c0 + 3, 3)
        gather_start(c0 + 3, 3, 1)
        consume(c0 + 2, 2, 0)

        @pl.when(t < NCHUNK // 4 - 1)
        def _():
            idx_wait(c0 + 4, 0)
            gather_start(c0 + 4, 0, 0)

        consume(c0 + 3, 3, 1)
        return 0

    lax.fori_loop(0, NCHUNK // 4, body, 0)
    scat_wait((NCHUNK - 2) % 4, 0)
    scat_wait((NCHUNK - 1) % 4, 1)
    t_wait((NCHUNK - 2) % 4, 0)
    t_wait((NCHUNK - 1) % 4, 1)
    plsc.subcore_barrier()
    pltpu.sync_copy(acc_s.at[pl.ds(sid * STRIPE, STRIPE)],
                    acc_out.at[cid, pl.ds(sid * STRIPE, STRIPE)])
    pltpu.sync_copy(s_s.at[pl.ds(sid * STRIPE, STRIPE)],
                    s_out.at[cid, pl.ds(sid * STRIPE, STRIPE)])


def _final_body(accp_ref, g_ref, st_ref, dinv_ref, b1_ref, w2_ref, b2_ref,
                out_ref, vacc_ref):
    i = pl.program_id(0)
    dinv = dinv_ref[...]
    s = st_ref[:, 0:1] + st_ref[:, 1:2]
    c = dinv * s + dinv * dinv
    ids = i * RB + lax.broadcasted_iota(jnp.int32, (RB, 1), 0)
    c = jnp.where(ids < N_NODES, c, 0.0)
    acc = accp_ref[0] + accp_ref[1] + g_ref[...]
    r = jnp.maximum(dinv * acc + b1_ref[...], 0.0)
    part = jnp.sum(c * r, axis=0, keepdims=True)

    @pl.when(i == 0)
    def _():
        vacc_ref[...] = part

    @pl.when(i > 0)
    def _():
        vacc_ref[...] += part

    @pl.when(i == NP // RB - 1)
    def _():
        v = vacc_ref[...]
        out_ref[...] = (
            jax.lax.dot_general(
                v, w2_ref[...], (((1,), (0,)), ((), ())),
                preferred_element_type=jnp.float32,
                precision=lax.Precision.HIGHEST,
            ) / float(N_NODES) + b2_ref[...]
        )


def _run_final(accp, g, st, dinv, b1, W2, b2):
    return pl.pallas_call(
        _final_body,
        grid=(NP // RB,),
        in_specs=[
            pl.BlockSpec((2, RB, 128), lambda i: (0, i, 0)),
            pl.BlockSpec((RB, 128), lambda i: (i, 0)),
            pl.BlockSpec((RB, 2), lambda i: (i, 0)),
            pl.BlockSpec((RB, 1), lambda i: (i, 0)),
            pl.BlockSpec((1, 128), lambda i: (0, 0)),
            pl.BlockSpec((128, 128), lambda i: (0, 0)),
            pl.BlockSpec((1, 128), lambda i: (0, 0)),
        ],
        out_specs=pl.BlockSpec((1, 128), lambda i: (0, 0)),
        out_shape=jax.ShapeDtypeStruct((1, 128), jnp.float32),
        scratch_shapes=[pltpu.VMEM((1, 128), jnp.float32)],
    )(accp, g, st, dinv, b1, W2, b2)


def kernel(x, edge_index, W1, b1, W2, b2):
    ei = edge_index.astype(jnp.int32)
    pad = N_NODES + (jnp.arange(EP - E, dtype=jnp.int32) % N_PAD_ROWS)
    ei2 = jnp.concatenate(
        [ei, jnp.broadcast_to(pad[None, :], (2, EP - E))], axis=1)
    x_p = jnp.concatenate([x, jnp.zeros((NP - N_NODES, 128), x.dtype)])

    h = _run_matmul(x_p, W1)
    degp = _deg_kernel(ei2)
    degt = degp.reshape(2, NP).T.reshape(NP, 2)
    g, dinv = _run_scale(h, degt)
    accp, sp = _agg_kernel(ei2, g, dinv.reshape(NP))
    st = sp.T.reshape(NP, 2)
    out = _run_final(accp, g, st, dinv, b1.reshape(1, 128), W2,
                     b2.reshape(1, 128))
    return out.reshape(128)


# speedup vs baseline: 55.3401x; 1.0299x over previous
import functools

import jax
import jax.numpy as jnp
from jax import lax
from jax.experimental import pallas as pl
from jax.experimental.pallas import tpu as pltpu
from jax.experimental.pallas import tpu_sc as plsc

N_NODES = 10000
NP = 10240
N_PAD_ROWS = NP - N_NODES
E = 320000
EP = 327680
NW = 32
CH = 128
NCHUNK = 80
EPW = NCHUNK * CH
STRIPE = NP // 16

_mesh = plsc.VectorSubcoreMesh(core_axis_name="c", subcore_axis_name="s")


def _zero_1d(ref, n):
    z = jnp.zeros((16,), jnp.float32)

    def body(i, _):
        ref[pl.ds(i * 16, 16)] = z
        return 0

    lax.fori_loop(0, n // 16, body, 0)


def _zero_2d(ref, rows):
    z = jnp.zeros((16,), jnp.float32)

    def body(i, _):
        for k in range(8):
            ref[i, pl.ds(k * 16, 16)] = z
        return 0

    lax.fori_loop(0, rows, body, 0)


@functools.partial(
    pl.kernel,
    out_type=jax.ShapeDtypeStruct((2, NP // 128, 128), jnp.float32),
    mesh=_mesh,
    scratch_types=[
        pltpu.VMEM((EPW,), jnp.int32),
        pltpu.VMEM((NP // 128, 128), jnp.float32),
        pltpu.VMEM((NP // 128,), jnp.int32),
        pltpu.VMEM_SHARED((NP // 128, 128), jnp.float32),
    ],
    compiler_params=pltpu.CompilerParams(needs_layout_passes=False),
)
def _deg_kernel(ei_hbm, deg_out, dst_v, hist_v, iota_v, deg_s):
    cid = lax.axis_index("c")
    sid = lax.axis_index("s")
    wid = sid * 2 + cid
    NR = NP // 128
    pltpu.sync_copy(ei_hbm.at[1, pl.ds(wid * EPW, EPW)], dst_v)
    _zero_2d(hist_v, NR)
    lane = lax.iota(jnp.int32, 16)
    for k in range(NR // 16):
        iota_v[pl.ds(k * 16, 16)] = lane + k * 16
    @pl.when(sid < NR // 8)
    def _():
        pltpu.sync_copy(hist_v.at[pl.ds(sid * 8, 8)],
                        deg_s.at[pl.ds(sid * 8, 8)])

    one = jnp.ones((16,), jnp.float32)

    def body(j, _):
        di = dst_v[pl.ds(j * 16, 16)]
        plsc.addupdate_scatter(hist_v, [di >> 7, di & 127], one)
        return 0

    lax.fori_loop(0, EPW // 16, body, 0)
    plsc.subcore_barrier()
    pltpu.sync_copy(hist_v, deg_s.at[iota_v], add=True)
    plsc.subcore_barrier()

    @pl.when(sid < NR // 8)
    def _():
        pltpu.sync_copy(deg_s.at[pl.ds(sid * 8, 8)],
                        deg_out.at[cid, pl.ds(sid * 8, 8)])


RB = 1024


def _scale_body(x_ref, degt_ref, g_ref, dinv_ref):
    deg = degt_ref[:, 0:1] + degt_ref[:, 1:2] + 1.0
    dinv = lax.rsqrt(deg)
    g_ref[...] = dinv * x_ref[...]
    dinv_ref[...] = dinv


def _run_scale(x_p, degt):
    return pl.pallas_call(
        _scale_body,
        grid=(NP // RB,),
        in_specs=[
            pl.BlockSpec((RB, 128), lambda i: (i, 0)),
            pl.BlockSpec((RB, 2), lambda i: (i, 0)),
        ],
        out_specs=[
            pl.BlockSpec((RB, 128), lambda i: (i, 0)),
            pl.BlockSpec((RB, 1), lambda i: (i, 0)),
        ],
        out_shape=[
            jax.ShapeDtypeStruct((NP, 128), jnp.float32),
            jax.ShapeDtypeStruct((NP, 1), jnp.float32),
        ],
    )(x_p, degt)


@functools.partial(
    pl.kernel,
    out_type=(
        jax.ShapeDtypeStruct((2, NP, 128), jnp.float32),
        jax.ShapeDtypeStruct((2, NP), jnp.float32),
    ),
    mesh=_mesh,
    scratch_types=[
        pltpu.VMEM((4, 2, CH), jnp.int32),
        pltpu.VMEM((NP,), jnp.float32),
        pltpu.VMEM((CH,), jnp.float32),
        pltpu.VMEM((CH,), jnp.float32),
        pltpu.VMEM((CH, 128), jnp.float32),
        pltpu.VMEM((CH, 128), jnp.float32),
        pltpu.VMEM_SHARED((NP, 128), jnp.float32),
        pltpu.VMEM_SHARED((NP,), jnp.float32),
        pltpu.SemaphoreType.DMA,
        pltpu.SemaphoreType.DMA,
        pltpu.SemaphoreType.DMA,
        pltpu.SemaphoreType.DMA,
        pltpu.SemaphoreType.DMA,
        pltpu.SemaphoreType.DMA,
        pltpu.SemaphoreType.DMA,
        pltpu.SemaphoreType.DMA,
        pltpu.SemaphoreType.DMA,
        pltpu.SemaphoreType.DMA,
    ],
    compiler_params=pltpu.CompilerParams(needs_layout_passes=False),
)
def _agg_kernel(ei_hbm, g_hbm, dinv_hbm,
                acc_out, s_out,
                sd_v, dinv_v, sval0_v, sval1_v, rows0_v, rows1_v,
                acc_s, s_s,
                isem0, isem1, isem2, isem3,
                gsem0, gsem1, ssem0, ssem1, tsem0, tsem1):
    cid = lax.axis_index("c")
    sid = lax.axis_index("s")
    wid = sid * 2 + cid
    isems = (isem0, isem1, isem2, isem3)
    rows = (rows0_v, rows1_v)
    svals = (sval0_v, sval1_v)
    gsems = (gsem0, gsem1)
    ssems = (ssem0, ssem1)
    tsems = (tsem0, tsem1)
    pltpu.sync_copy(dinv_hbm, dinv_v)
    _zero_2d(rows0_v, CH)
    for t in range(STRIPE // CH):
        pltpu.sync_copy(rows0_v, acc_s.at[pl.ds(sid * STRIPE + t * CH, CH)])
    _zero_1d(sval0_v, CH)
    for t in range(STRIPE // CH):
        pltpu.sync_copy(sval0_v, s_s.at[pl.ds(sid * STRIPE + t * CH, CH)])
    plsc.subcore_barrier()

    def idx_fetch(c, slot):
        pltpu.async_copy(ei_hbm.at[:, pl.ds(wid * EPW + c * CH, CH)],
                         sd_v.at[slot], isems[slot])

    def idx_wait(c, slot):
        pltpu.make_async_copy(ei_hbm.at[:, pl.ds(wid * EPW + c * CH, CH)],
                              sd_v.at[slot], isems[slot]).wait()

    def scat_wait(slot, buf):
        pltpu.make_async_copy(rows[buf], acc_s.at[sd_v.at[slot, 1]],
                              ssems[buf]).wait()

    def t_wait(slot, buf):
        pltpu.make_async_copy(svals[buf], s_s.at[sd_v.at[slot, 0]],
                              tsems[buf]).wait()

    def gather_start(c, slot, buf):
        @pl.when(c >= 2)
        def _():
            scat_wait((slot + 2) % 4, buf)

        pltpu.async_copy(g_hbm.at[sd_v.at[slot, 0]], rows[buf], gsems[buf])

    def gather_wait(c, slot, buf):
        pltpu.make_async_copy(g_hbm.at[sd_v.at[slot, 0]], rows[buf],
                              gsems[buf]).wait()

    def s_part(slot, buf):
        for k in range(CH // 16):
            di = sd_v[slot, 1, pl.ds(k * 16, 16)]
            svals[buf][pl.ds(k * 16, 16)] = plsc.load_gather(dinv_v, [di])

    def consume(c, slot, buf):
        @pl.when(c >= 2)
        def _():
            t_wait((slot + 2) % 4, buf)

        @pl.when(c + 2 < NCHUNK)
        def _():
            idx_fetch(c + 2, (slot + 2) % 4)

        s_part(slot, buf)
        gather_wait(c, slot, buf)
        pltpu.async_copy(rows[buf], acc_s.at[sd_v.at[slot, 1]], ssems[buf],
                         add=True)
        pltpu.async_copy(svals[buf], s_s.at[sd_v.at[slot, 0]], tsems[buf],
                         add=True)

    idx_fetch(0, 0)
    idx_fetch(1, 1)
    idx_wait(0, 0)
    gather_start(0, 0, 0)

    def body(t, _):
        c0 = 4 * t
        idx_wait(c0 + 1, 1)
        gather_start(c0 + 1, 1, 1)
        consume(c0, 0, 0)
        idx_wait(c0 + 2, 2)
        gather_start(c0 + 2, 2, 0)
        consume(c0 + 1, 1, 1)
        idx_wait(c0 + 3, 3)
        gather_start(c0 + 3, 3, 1)
        consume(c0 + 2, 2, 0)

        @pl.when(t < NCHUNK // 4 - 1)
        def _():
            idx_wait(c0 + 4, 0)
            gather_start(c0 + 4, 0, 0)

        consume(c0 + 3, 3, 1)
        return 0

    lax.fori_loop(0, NCHUNK // 4, body, 0)
    scat_wait((NCHUNK - 2) % 4, 0)
    scat_wait((NCHUNK - 1) % 4, 1)
    t_wait((NCHUNK - 2) % 4, 0)
    t_wait((NCHUNK - 1) % 4, 1)
    plsc.subcore_barrier()
    pltpu.sync_copy(acc_s.at[pl.ds(sid * STRIPE, STRIPE)],
                    acc_out.at[cid, pl.ds(sid * STRIPE, STRIPE)])
    pltpu.sync_copy(s_s.at[pl.ds(sid * STRIPE, STRIPE)],
                    s_out.at[cid, pl.ds(sid * STRIPE, STRIPE)])


def _final_body(accp_ref, g_ref, st_ref, dinv_ref, w1_ref, b1_ref, w2_ref,
                b2_ref, out_ref, vacc_ref):
    i = pl.program_id(0)
    dinv = dinv_ref[...]
    s = st_ref[:, 0:1] + st_ref[:, 1:2]
    c = dinv * s + dinv * dinv
    ids = i * RB + lax.broadcasted_iota(jnp.int32, (RB, 1), 0)
    c = jnp.where(ids < N_NODES, c, 0.0)
    agg = dinv * (accp_ref[0] + accp_ref[1] + g_ref[...])
    h = jax.lax.dot_general(
        agg, w1_ref[...], (((1,), (0,)), ((), ())),
        preferred_element_type=jnp.float32,
        precision=lax.Precision.HIGHEST,
    )
    r = jnp.maximum(h + b1_ref[...], 0.0)
    part = jnp.sum(c * r, axis=0, keepdims=True)

    @pl.when(i == 0)
    def _():
        vacc_ref[...] = part

    @pl.when(i > 0)
    def _():
        vacc_ref[...] += part

    @pl.when(i == NP // RB - 1)
    def _():
        v = vacc_ref[...]
        out_ref[...] = (
            jax.lax.dot_general(
                v, w2_ref[...], (((1,), (0,)), ((), ())),
                preferred_element_type=jnp.float32,
                precision=lax.Precision.HIGHEST,
            ) / float(N_NODES) + b2_ref[...]
        )


def _run_final(accp, g, st, dinv, W1, b1, W2, b2):
    return pl.pallas_call(
        _final_body,
        grid=(NP // RB,),
        in_specs=[
            pl.BlockSpec((2, RB, 128), lambda i: (0, i, 0)),
            pl.BlockSpec((RB, 128), lambda i: (i, 0)),
            pl.BlockSpec((RB, 2), lambda i: (i, 0)),
            pl.BlockSpec((RB, 1), lambda i: (i, 0)),
            pl.BlockSpec((128, 128), lambda i: (0, 0)),
            pl.BlockSpec((1, 128), lambda i: (0, 0)),
            pl.BlockSpec((128, 128), lambda i: (0, 0)),
            pl.BlockSpec((1, 128), lambda i: (0, 0)),
        ],
        out_specs=pl.BlockSpec((1, 128), lambda i: (0, 0)),
        out_shape=jax.ShapeDtypeStruct((1, 128), jnp.float32),
        scratch_shapes=[pltpu.VMEM((1, 128), jnp.float32)],
    )(accp, g, st, dinv, W1, b1, W2, b2)


def kernel(x, edge_index, W1, b1, W2, b2):
    ei = edge_index.astype(jnp.int32)
    pad = N_NODES + (jnp.arange(EP - E, dtype=jnp.int32) % N_PAD_ROWS)
    ei2 = jnp.concatenate(
        [ei, jnp.broadcast_to(pad[None, :], (2, EP - E))], axis=1)
    x_p = jnp.concatenate([x, jnp.zeros((NP - N_NODES, 128), x.dtype)])

    degp = _deg_kernel(ei2)
    degt = degp.reshape(2, NP).T.reshape(NP, 2)
    g, dinv = _run_scale(x_p, degt)
    accp, sp = _agg_kernel(ei2, g, dinv.reshape(NP))
    st = sp.T.reshape(NP, 2)
    out = _run_final(accp, g, st, dinv, W1, b1.reshape(1, 128), W2,
                     b2.reshape(1, 128))
    return out.reshape(128)
